# Initial kernel scaffold; baseline (speedup 1.0000x reference)
import jax
import jax.numpy as jnp
from jax.experimental import pallas as pl


def kernel(pts, mlp1_w1, mlp1_b1, mlp1_w2, mlp1_b2, mlp1_w3, mlp1_b3, lin1_w, lin1_b, conv1_w, conv1_b, lin2_w, lin2_b, conv2_w, conv2_b, mlp2_w1, mlp2_b1, mlp2_w2, mlp2_b2, f1_w1, f1_b1, f1_w2, f1_b2, f1_w3, f1_b3, f2_w1, f2_b1, f2_w2, f2_b2, f2_w3, f2_b3):
    raise NotImplementedError("write your pallas kernel here")


# speedup vs baseline: 7.4437x; 7.4437x over previous
import functools
import itertools

import numpy as np
import jax
import jax.numpy as jnp
from jax import lax
from jax.experimental import pallas as pl
from jax.experimental.pallas import tpu as pltpu

B, N, K, M = 8, 2048, 16, 2025
RB = 256
NB = 512
MB = 512


def _knn_body(pts_full_ref, pts_row_ref, idxf_ref, idx01_ref):
    b = pl.program_id(0)
    P = pts_full_ref[0]
    R = pts_row_ref[0]
    G = lax.dot_general(R, P, (((1,), (1,)), ((), ())),
                        preferred_element_type=jnp.float32)
    inner = -2.0 * G
    xxr = jnp.sum(R * R, axis=1, keepdims=True)
    ones = jnp.ones((1, 3), jnp.float32)
    xxc = lax.dot_general(ones, P * P, (((1,), (1,)), ((), ())),
                          preferred_element_type=jnp.float32)
    pd = (-xxr - inner) - xxc
    iota = lax.broadcasted_iota(jnp.int32, pd.shape, 1)
    cols = []
    for k in range(K):
        m = jnp.max(pd, axis=1, keepdims=True)
        sel = jnp.where(pd == m, iota, N)
        col = jnp.min(sel, axis=1)
        cols.append(col)
        if k < K - 1:
            pd = jnp.where(iota == col[:, None], -jnp.inf, pd)
    idx = jnp.concatenate([c[:, None] for c in cols], axis=1)
    idxf_ref[0] = idx + b * N
    idx01_ref[0, 0, :] = cols[0]
    idx01_ref[0, 1, :] = cols[1]


def _knn(pts):
    return pl.pallas_call(
        _knn_body,
        grid=(B, N // RB),
        in_specs=[pl.BlockSpec((1, N, 3), lambda b, i: (b, 0, 0)),
                  pl.BlockSpec((1, RB, 3), lambda b, i: (b, i, 0))],
        out_specs=[pl.BlockSpec((1, RB, K), lambda b, i: (b, i, 0)),
                   pl.BlockSpec((1, 2, RB), lambda b, i: (b, 0, i))],
        out_shape=[jax.ShapeDtypeStruct((B, N, K), jnp.int32),
                   jax.ShapeDtypeStruct((B, 2, N), jnp.int32)],
    )(pts, pts)


def _cov9(pts, idx01):
    nb0 = jnp.take_along_axis(pts, idx01[:, 0, :, None], axis=1)
    nb1 = jnp.take_along_axis(pts, idx01[:, 1, :, None], axis=1)
    cov = nb0[:, :, :, None] * nb1[:, :, None, :]
    return cov.reshape(B, N, 9).transpose(0, 2, 1)


def _maxpool(featT, idxf):
    D = featT.shape[-1]
    flat = featT.reshape(B * N, D)
    g = flat[idxf.reshape(-1)].reshape(B, N, K, D)
    return jnp.max(g, axis=2)


def _mlp1_body(pts_ref, cov_ref, w1p_ref, w1c_ref, b1_ref, w2_ref, b2_ref,
               w3_ref, b3_ref, out_ref):
    p = pts_ref[0]
    c = cov_ref[0]
    h = (lax.dot_general(p, w1p_ref[...], (((1,), (1,)), ((), ())),
                         preferred_element_type=jnp.float32)
         + lax.dot_general(c, w1c_ref[...], (((0,), (1,)), ((), ())),
                           preferred_element_type=jnp.float32))
    h = jax.nn.relu(h + b1_ref[...])
    h = jax.nn.relu(lax.dot_general(h, w2_ref[...], (((1,), (1,)), ((), ())),
                                    preferred_element_type=jnp.float32) + b2_ref[...])
    h = jax.nn.relu(lax.dot_general(h, w3_ref[...], (((1,), (1,)), ((), ())),
                                    preferred_element_type=jnp.float32) + b3_ref[...])
    out_ref[0] = h


def _mlp1(pts, cov9, w1, b1, w2, b2, w3, b3):
    w1p, w1c = w1[:, :3], w1[:, 3:]
    full = lambda shape: pl.BlockSpec(shape, lambda b, i: tuple(0 for _ in shape))
    return pl.pallas_call(
        _mlp1_body,
        grid=(B, N // NB),
        in_specs=[pl.BlockSpec((1, NB, 3), lambda b, i: (b, i, 0)),
                  pl.BlockSpec((1, 9, NB), lambda b, i: (b, 0, i)),
                  full((64, 3)), full((64, 9)), full((64,)),
                  full((64, 64)), full((64,)), full((64, 64)), full((64,))],
        out_specs=pl.BlockSpec((1, NB, 64), lambda b, i: (b, i, 0)),
        out_shape=jax.ShapeDtypeStruct((B, N, 64), jnp.float32),
    )(pts, cov9, w1p, w1c, b1, w2, b2, w3, b3)


def _lin_conv_body(x_ref, lw_ref, lb_ref, cw_ref, cb_ref, out_ref):
    x = x_ref[0]
    t = lax.dot_general(x, lw_ref[...], (((1,), (1,)), ((), ())),
                        preferred_element_type=jnp.float32) + lb_ref[...]
    h = jax.nn.relu(lax.dot_general(t, cw_ref[...], (((1,), (1,)), ((), ())),
                                    preferred_element_type=jnp.float32) + cb_ref[...])
    out_ref[0] = h


def _lin_conv(x, lw, lb, cw, cb):
    Din, Dout = lw.shape[1], cw.shape[0]
    full = lambda shape: pl.BlockSpec(shape, lambda b, i: tuple(0 for _ in shape))
    return pl.pallas_call(
        _lin_conv_body,
        grid=(B, N // NB),
        in_specs=[pl.BlockSpec((1, NB, Din), lambda b, i: (b, i, 0)),
                  full(lw.shape), full(lb.shape), full(cw.shape), full(cb.shape)],
        out_specs=pl.BlockSpec((1, NB, Dout), lambda b, i: (b, i, 0)),
        out_shape=jax.ShapeDtypeStruct((B, N, Dout), jnp.float32),
    )(x, lw, lb, cw, cb)


def _tail_body(x_ref, lw_ref, lb_ref, cw_ref, cb_ref, m1w_ref, m1b_ref,
               m2w_ref, m2b_ref, emb_ref, acc_ref):
    j = pl.program_id(1)
    x = x_ref[0]
    t = lax.dot_general(x, lw_ref[...], (((1,), (1,)), ((), ())),
                        preferred_element_type=jnp.float32) + lb_ref[...]
    y = lax.dot_general(t, cw_ref[...], (((1,), (1,)), ((), ())),
                        preferred_element_type=jnp.float32) + cb_ref[...]
    bm = jnp.max(y, axis=0, keepdims=True)

    @pl.when(j == 0)
    def _():
        acc_ref[...] = bm

    @pl.when(j > 0)
    def _():
        acc_ref[...] = jnp.maximum(acc_ref[...], bm)

    @pl.when(j == pl.num_programs(1) - 1)
    def _():
        g = acc_ref[...]
        g1 = jax.nn.relu(
            lax.dot_general(g, m1w_ref[...], (((1,), (1,)), ((), ())),
                            preferred_element_type=jnp.float32) + m1b_ref[...])
        g2 = lax.dot_general(g1, m2w_ref[...], (((1,), (1,)), ((), ())),
                             preferred_element_type=jnp.float32) + m2b_ref[...]
        emb_ref[...] = g2[None]


def _tail(x, lw, lb, cw, cb, m1w, m1b, m2w, m2b):
    full = lambda shape: pl.BlockSpec(shape, lambda b, i: tuple(0 for _ in shape))
    return pl.pallas_call(
        _tail_body,
        grid=(B, N // NB),
        in_specs=[pl.BlockSpec((1, NB, 128), lambda b, i: (b, i, 0)),
                  full(lw.shape), full(lb.shape), full(cw.shape), full(cb.shape),
                  full(m1w.shape), full(m1b.shape), full(m2w.shape), full(m2b.shape)],
        out_specs=pl.BlockSpec((1, 1, 512), lambda b, i: (b, 0, 0)),
        out_shape=jax.ShapeDtypeStruct((B, 1, 512), jnp.float32),
        scratch_shapes=[pltpu.VMEM((1, 1024), jnp.float32)],
    )(x, lw, lb, cw, cb, m1w, m1b, m2w, m2b)


def _dec_body(emb_ref, grid_ref, w1g_ref, e1w_ref, b1_ref, w2_ref, b2_ref,
              w3_ref, b3_ref, v1f_ref, e2w_ref, c1_ref, v2_ref, c2_ref,
              v3_ref, c3_ref, out_ref, fold1_ref):
    emb = emb_ref[0]
    e1 = lax.dot_general(emb, e1w_ref[...], (((1,), (1,)), ((), ())),
                         preferred_element_type=jnp.float32)
    e2 = lax.dot_general(emb, e2w_ref[...], (((1,), (1,)), ((), ())),
                         preferred_element_type=jnp.float32)
    g = grid_ref[...]
    t = jax.nn.relu(
        lax.dot_general(g, w1g_ref[...], (((1,), (1,)), ((), ())),
                        preferred_element_type=jnp.float32) + e1 + b1_ref[...])
    t = jax.nn.relu(lax.dot_general(t, w2_ref[...], (((1,), (1,)), ((), ())),
                                    preferred_element_type=jnp.float32) + b2_ref[...])
    f1 = lax.dot_general(t, w3_ref[...], (((1,), (1,)), ((), ())),
                         preferred_element_type=jnp.float32) + b3_ref[...]
    u = jax.nn.relu(
        lax.dot_general(f1, v1f_ref[...], (((1,), (1,)), ((), ())),
                        preferred_element_type=jnp.float32) + e2 + c1_ref[...])
    u = jax.nn.relu(lax.dot_general(u, v2_ref[...], (((1,), (1,)), ((), ())),
                                    preferred_element_type=jnp.float32) + c2_ref[...])
    f2 = lax.dot_general(u, v3_ref[...], (((1,), (1,)), ((), ())),
                         preferred_element_type=jnp.float32) + c3_ref[...]
    fold1_ref[0] = f1
    out_ref[0] = f2


@functools.lru_cache(maxsize=1)
def _grid_const():
    xs = np.linspace(-3, 3, 45)
    pts = np.array(list(itertools.product(xs, xs)), dtype=np.float32)
    return pts


def _decoder(emb, f1_w1, f1_b1, f1_w2, f1_b2, f1_w3, f1_b3,
             f2_w1, f2_b1, f2_w2, f2_b2, f2_w3, f2_b3):
    gridc = jnp.asarray(_grid_const())
    w1g = f1_w1[:, 512:514]
    e1w = f1_w1[:, :512]
    v1f = f2_w1[:, 512:515]
    e2w = f2_w1[:, :512]
    full = lambda shape: pl.BlockSpec(shape, lambda b, i: tuple(0 for _ in shape))
    return pl.pallas_call(
        _dec_body,
        grid=(B, (M + MB - 1) // MB),
        in_specs=[pl.BlockSpec((1, 1, 512), lambda b, i: (b, 0, 0)),
                  pl.BlockSpec((MB, 2), lambda b, i: (i, 0)),
                  full(w1g.shape), full(e1w.shape), full(f1_b1.shape),
                  full(f1_w2.shape), full(f1_b2.shape),
                  full(f1_w3.shape), full(f1_b3.shape),
                  full(v1f.shape), full(e2w.shape), full(f2_b1.shape),
                  full(f2_w2.shape), full(f2_b2.shape),
                  full(f2_w3.shape), full(f2_b3.shape)],
        out_specs=[pl.BlockSpec((1, MB, 3), lambda b, i: (b, i, 0)),
                   pl.BlockSpec((1, MB, 3), lambda b, i: (b, i, 0))],
        out_shape=[jax.ShapeDtypeStruct((B, M, 3), jnp.float32),
                   jax.ShapeDtypeStruct((B, M, 3), jnp.float32)],
    )(emb, gridc, w1g, e1w, f1_b1, f1_w2, f1_b2, f1_w3, f1_b3,
      v1f, e2w, f2_b1, f2_w2, f2_b2, f2_w3, f2_b3)


def kernel(pts, mlp1_w1, mlp1_b1, mlp1_w2, mlp1_b2, mlp1_w3, mlp1_b3,
           lin1_w, lin1_b, conv1_w, conv1_b, lin2_w, lin2_b,
           conv2_w, conv2_b, mlp2_w1, mlp2_b1, mlp2_w2, mlp2_b2,
           f1_w1, f1_b1, f1_w2, f1_b2, f1_w3, f1_b3,
           f2_w1, f2_b1, f2_w2, f2_b2, f2_w3, f2_b3):
    idxf, idx01 = _knn(pts)
    cov9 = _cov9(pts, idx01)
    h1 = _mlp1(pts, cov9, mlp1_w1, mlp1_b1, mlp1_w2, mlp1_b2, mlp1_w3, mlp1_b3)
    m1 = _maxpool(h1, idxf)
    h2 = _lin_conv(m1, lin1_w, lin1_b, conv1_w, conv1_b)
    m2 = _maxpool(h2, idxf)
    feat = _tail(m2, lin2_w, lin2_b, conv2_w, conv2_b,
                 mlp2_w1, mlp2_b1, mlp2_w2, mlp2_b2)
    output, fold1 = _decoder(feat, f1_w1, f1_b1, f1_w2, f1_b2, f1_w3, f1_b3,
                             f2_w1, f2_b1, f2_w2, f2_b2, f2_w3, f2_b3)
    return output, feat, feat.reshape(B, 512), fold1


# speedup vs baseline: 17.8815x; 2.4022x over previous
import functools
import itertools

import numpy as np
import jax
import jax.numpy as jnp
from jax import lax
from jax.experimental import pallas as pl
from jax.experimental.pallas import tpu as pltpu
from jax.experimental.pallas import tpu_sc as plsc

B, N, K, M = 8, 2048, 16, 2025
RB = 256
NB = 512
MB = 512


def _knn_body(pts_full_ref, pts_row_ref, idxf_ref, cov_ref):
    b = pl.program_id(0)
    P = pts_full_ref[0]
    R = pts_row_ref[0]
    G = lax.dot_general(R, P, (((1,), (1,)), ((), ())),
                        preferred_element_type=jnp.float32)
    inner = -2.0 * G
    xxr = jnp.sum(R * R, axis=1, keepdims=True)
    ones = jnp.ones((1, 3), jnp.float32)
    xxc = lax.dot_general(ones, P * P, (((1,), (1,)), ((), ())),
                          preferred_element_type=jnp.float32)
    pd = (-xxr - inner) - xxc
    iota = lax.broadcasted_iota(jnp.int32, pd.shape, 1)
    cols = []
    for k in range(K):
        m = jnp.max(pd, axis=1, keepdims=True)
        sel = jnp.where(pd == m, iota, N)
        col = jnp.min(sel, axis=1)
        cols.append(col)
        if k < K - 1:
            pd = jnp.where(iota == col[:, None], -jnp.inf, pd)
    idx = jnp.concatenate([c[:, None] for c in cols], axis=1)
    idxf_ref[0] = idx + b * N
    oh0 = (iota == cols[0][:, None]).astype(jnp.float32)
    oh1 = (iota == cols[1][:, None]).astype(jnp.float32)
    nb0 = lax.dot_general(oh0, P, (((1,), (0,)), ((), ())),
                          preferred_element_type=jnp.float32)
    nb1 = lax.dot_general(oh1, P, (((1,), (0,)), ((), ())),
                          preferred_element_type=jnp.float32)
    prods = [nb0[:, i:i + 1] * nb1[:, j:j + 1]
             for i in range(3) for j in range(3)]
    cov_ref[0] = jnp.concatenate(prods, axis=1)


def _knn(pts):
    return pl.pallas_call(
        _knn_body,
        grid=(B, N // RB),
        in_specs=[pl.BlockSpec((1, N, 3), lambda b, i: (b, 0, 0)),
                  pl.BlockSpec((1, RB, 3), lambda b, i: (b, i, 0))],
        out_specs=[pl.BlockSpec((1, RB, K), lambda b, i: (b, i, 0)),
                   pl.BlockSpec((1, RB, 9), lambda b, i: (b, i, 0))],
        out_shape=[jax.ShapeDtypeStruct((B, N, K), jnp.int32),
                   jax.ShapeDtypeStruct((B, N, 9), jnp.float32)],
    )(pts, pts)


def _sc_mesh():
    return plsc.VectorSubcoreMesh(core_axis_name="c", subcore_axis_name="s")


def _worker_id():
    nc = plsc.get_sparse_core_info().num_cores
    return lax.axis_index("s") * nc + lax.axis_index("c")


def _maxpool(featT, idxf):
    D = featT.shape[-1]
    Bn = B * N
    feat_flat = featT.reshape(Bn, D)
    idx_flat = idxf.reshape(Bn * K)
    NW = 32
    PW = Bn // NW
    CH = 8
    NCH = PW // CH

    @functools.partial(
        pl.kernel, mesh=_sc_mesh(),
        compiler_params=pltpu.CompilerParams(use_tc_tiling_on_sc=False),
        out_type=jax.ShapeDtypeStruct((Bn, D), jnp.float32),
        scratch_types=[pltpu.VMEM((CH * K,), jnp.int32),
                       pltpu.VMEM((CH * K, D), jnp.float32),
                       pltpu.VMEM((CH, D), jnp.float32),
                       pltpu.SemaphoreType.DMA])
    def k(feat_hbm, idx_hbm, out_hbm, idx_v, rows_v, out_v, sem):
        wid = _worker_id()
        base = wid * PW

        def chunk(c, carry):
            pt0 = base + c * CH
            pltpu.sync_copy(idx_hbm.at[pl.ds(pt0 * K, CH * K)], idx_v)
            pltpu.async_copy(feat_hbm.at[idx_v], rows_v, sem).wait()

            def per_point(p, carry2):
                for cc in range(D // 16):
                    sl = pl.ds(cc * 16, 16)
                    acc = rows_v[p * K, sl]
                    for r in range(1, K):
                        acc = jnp.maximum(acc, rows_v[p * K + r, sl])
                    out_v[p, sl] = acc
                return carry2

            lax.fori_loop(0, CH, per_point, 0)
            pltpu.sync_copy(out_v, out_hbm.at[pl.ds(pt0, CH)])
            return carry

        lax.fori_loop(0, NCH, chunk, 0)

    return k(feat_flat, idx_flat).reshape(B, N, D)


def _mlp1_body(pts_ref, cov_ref, w1p_ref, w1c_ref, b1_ref, w2_ref, b2_ref,
               w3_ref, b3_ref, out_ref):
    p = pts_ref[0]
    c = cov_ref[0]
    h = (lax.dot_general(p, w1p_ref[...], (((1,), (1,)), ((), ())),
                         preferred_element_type=jnp.float32)
         + lax.dot_general(c, w1c_ref[...], (((1,), (1,)), ((), ())),
                           preferred_element_type=jnp.float32))
    h = jax.nn.relu(h + b1_ref[...])
    h = jax.nn.relu(lax.dot_general(h, w2_ref[...], (((1,), (1,)), ((), ())),
                                    preferred_element_type=jnp.float32) + b2_ref[...])
    h = jax.nn.relu(lax.dot_general(h, w3_ref[...], (((1,), (1,)), ((), ())),
                                    preferred_element_type=jnp.float32) + b3_ref[...])
    out_ref[0] = h


def _mlp1(pts, cov9, w1, b1, w2, b2, w3, b3):
    w1p, w1c = w1[:, :3], w1[:, 3:]
    full = lambda shape: pl.BlockSpec(shape, lambda b, i: tuple(0 for _ in shape))
    return pl.pallas_call(
        _mlp1_body,
        grid=(B, N // NB),
        in_specs=[pl.BlockSpec((1, NB, 3), lambda b, i: (b, i, 0)),
                  pl.BlockSpec((1, NB, 9), lambda b, i: (b, i, 0)),
                  full((64, 3)), full((64, 9)), full((64,)),
                  full((64, 64)), full((64,)), full((64, 64)), full((64,))],
        out_specs=pl.BlockSpec((1, NB, 64), lambda b, i: (b, i, 0)),
        out_shape=jax.ShapeDtypeStruct((B, N, 64), jnp.float32),
    )(pts, cov9, w1p, w1c, b1, w2, b2, w3, b3)


def _lin_conv_body(x_ref, lw_ref, lb_ref, cw_ref, cb_ref, out_ref):
    x = x_ref[0]
    t = lax.dot_general(x, lw_ref[...], (((1,), (1,)), ((), ())),
                        preferred_element_type=jnp.float32) + lb_ref[...]
    h = jax.nn.relu(lax.dot_general(t, cw_ref[...], (((1,), (1,)), ((), ())),
                                    preferred_element_type=jnp.float32) + cb_ref[...])
    out_ref[0] = h


def _lin_conv(x, lw, lb, cw, cb):
    Din, Dout = lw.shape[1], cw.shape[0]
    full = lambda shape: pl.BlockSpec(shape, lambda b, i: tuple(0 for _ in shape))
    return pl.pallas_call(
        _lin_conv_body,
        grid=(B, N // NB),
        in_specs=[pl.BlockSpec((1, NB, Din), lambda b, i: (b, i, 0)),
                  full(lw.shape), full(lb.shape), full(cw.shape), full(cb.shape)],
        out_specs=pl.BlockSpec((1, NB, Dout), lambda b, i: (b, i, 0)),
        out_shape=jax.ShapeDtypeStruct((B, N, Dout), jnp.float32),
    )(x, lw, lb, cw, cb)


def _tail_body(x_ref, lw_ref, lb_ref, cw_ref, cb_ref, m1w_ref, m1b_ref,
               m2w_ref, m2b_ref, emb_ref, acc_ref):
    j = pl.program_id(1)
    x = x_ref[0]
    t = lax.dot_general(x, lw_ref[...], (((1,), (1,)), ((), ())),
                        preferred_element_type=jnp.float32) + lb_ref[...]
    y = lax.dot_general(t, cw_ref[...], (((1,), (1,)), ((), ())),
                        preferred_element_type=jnp.float32) + cb_ref[...]
    bm = jnp.max(y, axis=0, keepdims=True)

    @pl.when(j == 0)
    def _():
        acc_ref[...] = bm

    @pl.when(j > 0)
    def _():
        acc_ref[...] = jnp.maximum(acc_ref[...], bm)

    @pl.when(j == pl.num_programs(1) - 1)
    def _():
        g = acc_ref[...]
        g1 = jax.nn.relu(
            lax.dot_general(g, m1w_ref[...], (((1,), (1,)), ((), ())),
                            preferred_element_type=jnp.float32) + m1b_ref[...])
        g2 = lax.dot_general(g1, m2w_ref[...], (((1,), (1,)), ((), ())),
                             preferred_element_type=jnp.float32) + m2b_ref[...]
        emb_ref[...] = g2[None]


def _tail(x, lw, lb, cw, cb, m1w, m1b, m2w, m2b):
    full = lambda shape: pl.BlockSpec(shape, lambda b, i: tuple(0 for _ in shape))
    return pl.pallas_call(
        _tail_body,
        grid=(B, N // NB),
        in_specs=[pl.BlockSpec((1, NB, 128), lambda b, i: (b, i, 0)),
                  full(lw.shape), full(lb.shape), full(cw.shape), full(cb.shape),
                  full(m1w.shape), full(m1b.shape), full(m2w.shape), full(m2b.shape)],
        out_specs=pl.BlockSpec((1, 1, 512), lambda b, i: (b, 0, 0)),
        out_shape=jax.ShapeDtypeStruct((B, 1, 512), jnp.float32),
        scratch_shapes=[pltpu.VMEM((1, 1024), jnp.float32)],
    )(x, lw, lb, cw, cb, m1w, m1b, m2w, m2b)


def _dec_body(emb_ref, grid_ref, w1g_ref, e1w_ref, b1_ref, w2_ref, b2_ref,
              w3_ref, b3_ref, v1f_ref, e2w_ref, c1_ref, v2_ref, c2_ref,
              v3_ref, c3_ref, out_ref, fold1_ref):
    emb = emb_ref[0]
    e1 = lax.dot_general(emb, e1w_ref[...], (((1,), (1,)), ((), ())),
                         preferred_element_type=jnp.float32)
    e2 = lax.dot_general(emb, e2w_ref[...], (((1,), (1,)), ((), ())),
                         preferred_element_type=jnp.float32)
    g = grid_ref[...]
    t = jax.nn.relu(
        lax.dot_general(g, w1g_ref[...], (((1,), (1,)), ((), ())),
                        preferred_element_type=jnp.float32) + e1 + b1_ref[...])
    t = jax.nn.relu(lax.dot_general(t, w2_ref[...], (((1,), (1,)), ((), ())),
                                    preferred_element_type=jnp.float32) + b2_ref[...])
    f1 = lax.dot_general(t, w3_ref[...], (((1,), (1,)), ((), ())),
                         preferred_element_type=jnp.float32) + b3_ref[...]
    u = jax.nn.relu(
        lax.dot_general(f1, v1f_ref[...], (((1,), (1,)), ((), ())),
                        preferred_element_type=jnp.float32) + e2 + c1_ref[...])
    u = jax.nn.relu(lax.dot_general(u, v2_ref[...], (((1,), (1,)), ((), ())),
                                    preferred_element_type=jnp.float32) + c2_ref[...])
    f2 = lax.dot_general(u, v3_ref[...], (((1,), (1,)), ((), ())),
                         preferred_element_type=jnp.float32) + c3_ref[...]
    fold1_ref[0] = f1
    out_ref[0] = f2


@functools.lru_cache(maxsize=1)
def _grid_const():
    xs = np.linspace(-3, 3, 45)
    pts = np.array(list(itertools.product(xs, xs)), dtype=np.float32)
    return pts


def _decoder(emb, f1_w1, f1_b1, f1_w2, f1_b2, f1_w3, f1_b3,
             f2_w1, f2_b1, f2_w2, f2_b2, f2_w3, f2_b3):
    gridc = jnp.asarray(_grid_const())
    w1g = f1_w1[:, 512:514]
    e1w = f1_w1[:, :512]
    v1f = f2_w1[:, 512:515]
    e2w = f2_w1[:, :512]
    full = lambda shape: pl.BlockSpec(shape, lambda b, i: tuple(0 for _ in shape))
    return pl.pallas_call(
        _dec_body,
        grid=(B, (M + MB - 1) // MB),
        in_specs=[pl.BlockSpec((1, 1, 512), lambda b, i: (b, 0, 0)),
                  pl.BlockSpec((MB, 2), lambda b, i: (i, 0)),
                  full(w1g.shape), full(e1w.shape), full(f1_b1.shape),
                  full(f1_w2.shape), full(f1_b2.shape),
                  full(f1_w3.shape), full(f1_b3.shape),
                  full(v1f.shape), full(e2w.shape), full(f2_b1.shape),
                  full(f2_w2.shape), full(f2_b2.shape),
                  full(f2_w3.shape), full(f2_b3.shape)],
        out_specs=[pl.BlockSpec((1, MB, 3), lambda b, i: (b, i, 0)),
                   pl.BlockSpec((1, MB, 3), lambda b, i: (b, i, 0))],
        out_shape=[jax.ShapeDtypeStruct((B, M, 3), jnp.float32),
                   jax.ShapeDtypeStruct((B, M, 3), jnp.float32)],
    )(emb, gridc, w1g, e1w, f1_b1, f1_w2, f1_b2, f1_w3, f1_b3,
      v1f, e2w, f2_b1, f2_w2, f2_b2, f2_w3, f2_b3)


def kernel(pts, mlp1_w1, mlp1_b1, mlp1_w2, mlp1_b2, mlp1_w3, mlp1_b3,
           lin1_w, lin1_b, conv1_w, conv1_b, lin2_w, lin2_b,
           conv2_w, conv2_b, mlp2_w1, mlp2_b1, mlp2_w2, mlp2_b2,
           f1_w1, f1_b1, f1_w2, f1_b2, f1_w3, f1_b3,
           f2_w1, f2_b1, f2_w2, f2_b2, f2_w3, f2_b3):
    idxf, cov9 = _knn(pts)
    h1 = _mlp1(pts, cov9, mlp1_w1, mlp1_b1, mlp1_w2, mlp1_b2, mlp1_w3, mlp1_b3)
    m1 = _maxpool(h1, idxf)
    h2 = _lin_conv(m1, lin1_w, lin1_b, conv1_w, conv1_b)
    m2 = _maxpool(h2, idxf)
    feat = _tail(m2, lin2_w, lin2_b, conv2_w, conv2_b,
                 mlp2_w1, mlp2_b1, mlp2_w2, mlp2_b2)
    output, fold1 = _decoder(feat, f1_w1, f1_b1, f1_w2, f1_b2, f1_w3, f1_b3,
                             f2_w1, f2_b1, f2_w2, f2_b2, f2_w3, f2_b3)
    return output, feat, feat.reshape(B, 512), fold1


# speedup vs baseline: 20.3089x; 1.1357x over previous
import functools
import itertools

import numpy as np
import jax
import jax.numpy as jnp
from jax import lax
from jax.experimental import pallas as pl
from jax.experimental.pallas import tpu as pltpu
from jax.experimental.pallas import tpu_sc as plsc

B, N, K, M = 8, 2048, 16, 2025
RB = 256
NB = 512
MB = 512


def _knn_body(pts_full_ref, pts_row_ref, idxf_ref, cov_ref):
    b = pl.program_id(0)
    P = pts_full_ref[0]
    R = pts_row_ref[0]
    G = lax.dot_general(R, P, (((1,), (1,)), ((), ())),
                        preferred_element_type=jnp.float32)
    inner = -2.0 * G
    xxr = jnp.sum(R * R, axis=1, keepdims=True)
    ones = jnp.ones((1, 3), jnp.float32)
    xxc = lax.dot_general(ones, P * P, (((1,), (1,)), ((), ())),
                          preferred_element_type=jnp.float32)
    pd = (-xxr - inner) - xxc
    iota = lax.broadcasted_iota(jnp.int32, pd.shape, 1)
    cols = []
    for k in range(K):
        col = jnp.argmax(pd, axis=1).astype(jnp.int32)
        cols.append(col)
        if k < K - 1:
            pd = jnp.where(iota == col[:, None], -jnp.inf, pd)
    idx = jnp.concatenate([c[:, None] for c in cols], axis=1)
    idxf_ref[0] = idx + b * N
    oh0 = (iota == cols[0][:, None]).astype(jnp.float32)
    oh1 = (iota == cols[1][:, None]).astype(jnp.float32)
    nb0 = lax.dot_general(oh0, P, (((1,), (0,)), ((), ())),
                          preferred_element_type=jnp.float32)
    nb1 = lax.dot_general(oh1, P, (((1,), (0,)), ((), ())),
                          preferred_element_type=jnp.float32)
    prods = [nb0[:, i:i + 1] * nb1[:, j:j + 1]
             for i in range(3) for j in range(3)]
    cov_ref[0] = jnp.concatenate(prods, axis=1)


def _knn(pts):
    return pl.pallas_call(
        _knn_body,
        grid=(B, N // RB),
        in_specs=[pl.BlockSpec((1, N, 3), lambda b, i: (b, 0, 0)),
                  pl.BlockSpec((1, RB, 3), lambda b, i: (b, i, 0))],
        out_specs=[pl.BlockSpec((1, RB, K), lambda b, i: (b, i, 0)),
                   pl.BlockSpec((1, RB, 9), lambda b, i: (b, i, 0))],
        out_shape=[jax.ShapeDtypeStruct((B, N, K), jnp.int32),
                   jax.ShapeDtypeStruct((B, N, 9), jnp.float32)],
    )(pts, pts)


def _sc_mesh():
    return plsc.VectorSubcoreMesh(core_axis_name="c", subcore_axis_name="s")


def _worker_id():
    nc = plsc.get_sparse_core_info().num_cores
    return lax.axis_index("s") * nc + lax.axis_index("c")


def _maxpool(featT, idxf):
    D = featT.shape[-1]
    Bn = B * N
    feat_flat = featT.reshape(Bn, D)
    idx_flat = idxf.reshape(Bn * K)
    NW = 32
    PW = Bn // NW
    CH = 8
    NCH = PW // CH

    @functools.partial(
        pl.kernel, mesh=_sc_mesh(),
        compiler_params=pltpu.CompilerParams(use_tc_tiling_on_sc=False),
        out_type=jax.ShapeDtypeStruct((Bn, D), jnp.float32),
        scratch_types=[pltpu.VMEM((CH * K,), jnp.int32),
                       pltpu.VMEM((CH * K, D), jnp.float32),
                       pltpu.VMEM((CH, D), jnp.float32),
                       pltpu.SemaphoreType.DMA])
    def k(feat_hbm, idx_hbm, out_hbm, idx_v, rows_v, out_v, sem):
        wid = _worker_id()
        base = wid * PW

        def chunk(c, carry):
            pt0 = base + c * CH
            pltpu.sync_copy(idx_hbm.at[pl.ds(pt0 * K, CH * K)], idx_v)
            pltpu.async_copy(feat_hbm.at[idx_v], rows_v, sem).wait()

            def per_point(p, carry2):
                for cc in range(D // 16):
                    sl = pl.ds(cc * 16, 16)
                    acc = rows_v[p * K, sl]
                    for r in range(1, K):
                        acc = jnp.maximum(acc, rows_v[p * K + r, sl])
                    out_v[p, sl] = acc
                return carry2

            lax.fori_loop(0, CH, per_point, 0)
            pltpu.sync_copy(out_v, out_hbm.at[pl.ds(pt0, CH)])
            return carry

        lax.fori_loop(0, NCH, chunk, 0)

    return k(feat_flat, idx_flat).reshape(B, N, D)


def _mlp1_body(pts_ref, cov_ref, w1p_ref, w1c_ref, b1_ref, w2_ref, b2_ref,
               w3_ref, b3_ref, out_ref):
    p = pts_ref[0]
    c = cov_ref[0]
    h = (lax.dot_general(p, w1p_ref[...], (((1,), (1,)), ((), ())),
                         preferred_element_type=jnp.float32)
         + lax.dot_general(c, w1c_ref[...], (((1,), (1,)), ((), ())),
                           preferred_element_type=jnp.float32))
    h = jax.nn.relu(h + b1_ref[...])
    h = jax.nn.relu(lax.dot_general(h, w2_ref[...], (((1,), (1,)), ((), ())),
                                    preferred_element_type=jnp.float32) + b2_ref[...])
    h = jax.nn.relu(lax.dot_general(h, w3_ref[...], (((1,), (1,)), ((), ())),
                                    preferred_element_type=jnp.float32) + b3_ref[...])
    out_ref[0] = h


def _mlp1(pts, cov9, w1, b1, w2, b2, w3, b3):
    w1p, w1c = w1[:, :3], w1[:, 3:]
    full = lambda shape: pl.BlockSpec(shape, lambda b, i: tuple(0 for _ in shape))
    return pl.pallas_call(
        _mlp1_body,
        grid=(B, N // NB),
        in_specs=[pl.BlockSpec((1, NB, 3), lambda b, i: (b, i, 0)),
                  pl.BlockSpec((1, NB, 9), lambda b, i: (b, i, 0)),
                  full((64, 3)), full((64, 9)), full((64,)),
                  full((64, 64)), full((64,)), full((64, 64)), full((64,))],
        out_specs=pl.BlockSpec((1, NB, 64), lambda b, i: (b, i, 0)),
        out_shape=jax.ShapeDtypeStruct((B, N, 64), jnp.float32),
    )(pts, cov9, w1p, w1c, b1, w2, b2, w3, b3)


def _lin_conv_body(x_ref, lw_ref, lb_ref, cw_ref, cb_ref, out_ref):
    x = x_ref[0]
    t = lax.dot_general(x, lw_ref[...], (((1,), (1,)), ((), ())),
                        preferred_element_type=jnp.float32) + lb_ref[...]
    h = jax.nn.relu(lax.dot_general(t, cw_ref[...], (((1,), (1,)), ((), ())),
                                    preferred_element_type=jnp.float32) + cb_ref[...])
    out_ref[0] = h


def _lin_conv(x, lw, lb, cw, cb):
    Din, Dout = lw.shape[1], cw.shape[0]
    full = lambda shape: pl.BlockSpec(shape, lambda b, i: tuple(0 for _ in shape))
    return pl.pallas_call(
        _lin_conv_body,
        grid=(B, N // NB),
        in_specs=[pl.BlockSpec((1, NB, Din), lambda b, i: (b, i, 0)),
                  full(lw.shape), full(lb.shape), full(cw.shape), full(cb.shape)],
        out_specs=pl.BlockSpec((1, NB, Dout), lambda b, i: (b, i, 0)),
        out_shape=jax.ShapeDtypeStruct((B, N, Dout), jnp.float32),
    )(x, lw, lb, cw, cb)


def _tail_body(x_ref, lw_ref, lb_ref, cw_ref, cb_ref, m1w_ref, m1b_ref,
               m2w_ref, m2b_ref, emb_ref, acc_ref):
    j = pl.program_id(1)
    x = x_ref[0]
    t = lax.dot_general(x, lw_ref[...], (((1,), (1,)), ((), ())),
                        preferred_element_type=jnp.float32) + lb_ref[...]
    y = lax.dot_general(t, cw_ref[...], (((1,), (1,)), ((), ())),
                        preferred_element_type=jnp.float32) + cb_ref[...]
    bm = jnp.max(y, axis=0, keepdims=True)

    @pl.when(j == 0)
    def _():
        acc_ref[...] = bm

    @pl.when(j > 0)
    def _():
        acc_ref[...] = jnp.maximum(acc_ref[...], bm)

    @pl.when(j == pl.num_programs(1) - 1)
    def _():
        g = acc_ref[...]
        g1 = jax.nn.relu(
            lax.dot_general(g, m1w_ref[...], (((1,), (1,)), ((), ())),
                            preferred_element_type=jnp.float32) + m1b_ref[...])
        g2 = lax.dot_general(g1, m2w_ref[...], (((1,), (1,)), ((), ())),
                             preferred_element_type=jnp.float32) + m2b_ref[...]
        emb_ref[...] = g2[None]


def _tail(x, lw, lb, cw, cb, m1w, m1b, m2w, m2b):
    full = lambda shape: pl.BlockSpec(shape, lambda b, i: tuple(0 for _ in shape))
    return pl.pallas_call(
        _tail_body,
        grid=(B, N // NB),
        in_specs=[pl.BlockSpec((1, NB, 128), lambda b, i: (b, i, 0)),
                  full(lw.shape), full(lb.shape), full(cw.shape), full(cb.shape),
                  full(m1w.shape), full(m1b.shape), full(m2w.shape), full(m2b.shape)],
        out_specs=pl.BlockSpec((1, 1, 512), lambda b, i: (b, 0, 0)),
        out_shape=jax.ShapeDtypeStruct((B, 1, 512), jnp.float32),
        scratch_shapes=[pltpu.VMEM((1, 1024), jnp.float32)],
    )(x, lw, lb, cw, cb, m1w, m1b, m2w, m2b)


def _dec_body(emb_ref, grid_ref, w1g_ref, e1w_ref, b1_ref, w2_ref, b2_ref,
              w3_ref, b3_ref, v1f_ref, e2w_ref, c1_ref, v2_ref, c2_ref,
              v3_ref, c3_ref, out_ref, fold1_ref):
    emb = emb_ref[0]
    e1 = lax.dot_general(emb, e1w_ref[...], (((1,), (1,)), ((), ())),
                         preferred_element_type=jnp.float32)
    e2 = lax.dot_general(emb, e2w_ref[...], (((1,), (1,)), ((), ())),
                         preferred_element_type=jnp.float32)
    g = grid_ref[...]
    t = jax.nn.relu(
        lax.dot_general(g, w1g_ref[...], (((1,), (1,)), ((), ())),
                        preferred_element_type=jnp.float32) + e1 + b1_ref[...])
    t = jax.nn.relu(lax.dot_general(t, w2_ref[...], (((1,), (1,)), ((), ())),
                                    preferred_element_type=jnp.float32) + b2_ref[...])
    f1 = lax.dot_general(t, w3_ref[...], (((1,), (1,)), ((), ())),
                         preferred_element_type=jnp.float32) + b3_ref[...]
    u = jax.nn.relu(
        lax.dot_general(f1, v1f_ref[...], (((1,), (1,)), ((), ())),
                        preferred_element_type=jnp.float32) + e2 + c1_ref[...])
    u = jax.nn.relu(lax.dot_general(u, v2_ref[...], (((1,), (1,)), ((), ())),
                                    preferred_element_type=jnp.float32) + c2_ref[...])
    f2 = lax.dot_general(u, v3_ref[...], (((1,), (1,)), ((), ())),
                         preferred_element_type=jnp.float32) + c3_ref[...]
    fold1_ref[0] = f1
    out_ref[0] = f2


@functools.lru_cache(maxsize=1)
def _grid_const():
    xs = np.linspace(-3, 3, 45)
    pts = np.array(list(itertools.product(xs, xs)), dtype=np.float32)
    return pts


def _decoder(emb, f1_w1, f1_b1, f1_w2, f1_b2, f1_w3, f1_b3,
             f2_w1, f2_b1, f2_w2, f2_b2, f2_w3, f2_b3):
    gridc = jnp.asarray(_grid_const())
    w1g = f1_w1[:, 512:514]
    e1w = f1_w1[:, :512]
    v1f = f2_w1[:, 512:515]
    e2w = f2_w1[:, :512]
    full = lambda shape: pl.BlockSpec(shape, lambda b, i: tuple(0 for _ in shape))
    return pl.pallas_call(
        _dec_body,
        grid=(B, (M + MB - 1) // MB),
        in_specs=[pl.BlockSpec((1, 1, 512), lambda b, i: (b, 0, 0)),
                  pl.BlockSpec((MB, 2), lambda b, i: (i, 0)),
                  full(w1g.shape), full(e1w.shape), full(f1_b1.shape),
                  full(f1_w2.shape), full(f1_b2.shape),
                  full(f1_w3.shape), full(f1_b3.shape),
                  full(v1f.shape), full(e2w.shape), full(f2_b1.shape),
                  full(f2_w2.shape), full(f2_b2.shape),
                  full(f2_w3.shape), full(f2_b3.shape)],
        out_specs=[pl.BlockSpec((1, MB, 3), lambda b, i: (b, i, 0)),
                   pl.BlockSpec((1, MB, 3), lambda b, i: (b, i, 0))],
        out_shape=[jax.ShapeDtypeStruct((B, M, 3), jnp.float32),
                   jax.ShapeDtypeStruct((B, M, 3), jnp.float32)],
    )(emb, gridc, w1g, e1w, f1_b1, f1_w2, f1_b2, f1_w3, f1_b3,
      v1f, e2w, f2_b1, f2_w2, f2_b2, f2_w3, f2_b3)


def kernel(pts, mlp1_w1, mlp1_b1, mlp1_w2, mlp1_b2, mlp1_w3, mlp1_b3,
           lin1_w, lin1_b, conv1_w, conv1_b, lin2_w, lin2_b,
           conv2_w, conv2_b, mlp2_w1, mlp2_b1, mlp2_w2, mlp2_b2,
           f1_w1, f1_b1, f1_w2, f1_b2, f1_w3, f1_b3,
           f2_w1, f2_b1, f2_w2, f2_b2, f2_w3, f2_b3):
    idxf, cov9 = _knn(pts)
    h1 = _mlp1(pts, cov9, mlp1_w1, mlp1_b1, mlp1_w2, mlp1_b2, mlp1_w3, mlp1_b3)
    m1 = _maxpool(h1, idxf)
    h2 = _lin_conv(m1, lin1_w, lin1_b, conv1_w, conv1_b)
    m2 = _maxpool(h2, idxf)
    feat = _tail(m2, lin2_w, lin2_b, conv2_w, conv2_b,
                 mlp2_w1, mlp2_b1, mlp2_w2, mlp2_b2)
    output, fold1 = _decoder(feat, f1_w1, f1_b1, f1_w2, f1_b2, f1_w3, f1_b3,
                             f2_w1, f2_b1, f2_w2, f2_b2, f2_w3, f2_b3)
    return output, feat, feat.reshape(B, 512), fold1


# speedup vs baseline: 22.9757x; 1.1313x over previous
import functools
import itertools

import numpy as np
import jax
import jax.numpy as jnp
from jax import lax
from jax.experimental import pallas as pl
from jax.experimental.pallas import tpu as pltpu
from jax.experimental.pallas import tpu_sc as plsc

B, N, K, M = 8, 2048, 16, 2025
RB = 256
NB = 512
MB = 512


def _knn_body(pts_full_ref, pts_row_ref, idxf_ref, cov_ref):
    b = pl.program_id(0)
    P = pts_full_ref[0]
    R = pts_row_ref[0]
    G = lax.dot_general(R, P, (((1,), (1,)), ((), ())),
                        preferred_element_type=jnp.float32)
    inner = -2.0 * G
    xxr = jnp.sum(R * R, axis=1, keepdims=True)
    ones = jnp.ones((1, 3), jnp.float32)
    xxc = lax.dot_general(ones, P * P, (((1,), (1,)), ((), ())),
                          preferred_element_type=jnp.float32)
    pd = (-xxr - inner) - xxc
    iota = lax.broadcasted_iota(jnp.int32, pd.shape, 1)
    cols = []
    for k in range(K):
        col = jnp.argmax(pd, axis=1).astype(jnp.int32)
        cols.append(col)
        if k < K - 1:
            pd = jnp.where(iota == col[:, None], -jnp.inf, pd)
    idx = jnp.concatenate([c[:, None] for c in cols], axis=1)
    idxf_ref[0] = idx + b * N
    oh0 = (iota == cols[0][:, None]).astype(jnp.float32)
    oh1 = (iota == cols[1][:, None]).astype(jnp.float32)
    nb0 = lax.dot_general(oh0, P, (((1,), (0,)), ((), ())),
                          preferred_element_type=jnp.float32)
    nb1 = lax.dot_general(oh1, P, (((1,), (0,)), ((), ())),
                          preferred_element_type=jnp.float32)
    prods = [nb0[:, i:i + 1] * nb1[:, j:j + 1]
             for i in range(3) for j in range(3)]
    cov_ref[0] = jnp.concatenate(prods, axis=1)


def _knn(pts):
    return pl.pallas_call(
        _knn_body,
        grid=(B, N // RB),
        in_specs=[pl.BlockSpec((1, N, 3), lambda b, i: (b, 0, 0)),
                  pl.BlockSpec((1, RB, 3), lambda b, i: (b, i, 0))],
        out_specs=[pl.BlockSpec((1, RB, K), lambda b, i: (b, i, 0)),
                   pl.BlockSpec((1, RB, 9), lambda b, i: (b, i, 0))],
        out_shape=[jax.ShapeDtypeStruct((B, N, K), jnp.int32),
                   jax.ShapeDtypeStruct((B, N, 9), jnp.float32)],
    )(pts, pts)


def _sc_mesh():
    return plsc.VectorSubcoreMesh(core_axis_name="c", subcore_axis_name="s")


def _worker_id():
    nc = plsc.get_sparse_core_info().num_cores
    return lax.axis_index("s") * nc + lax.axis_index("c")


def _maxpool(featT, idxf):
    D = featT.shape[-1]
    Bn = B * N
    feat_flat = featT.reshape(Bn, D)
    idx_flat = idxf.reshape(Bn * K)
    NW = 32
    PW = Bn // NW
    CH = 8
    Q = 4
    NI = PW // (CH * Q)

    rows_t = pltpu.VMEM((CH * K, D), jnp.float32)

    @functools.partial(
        pl.kernel, mesh=_sc_mesh(),
        compiler_params=pltpu.CompilerParams(use_tc_tiling_on_sc=False),
        out_type=jax.ShapeDtypeStruct((Bn, D), jnp.float32),
        scratch_types=[pltpu.VMEM((PW * K,), jnp.int32),
                       rows_t, rows_t, rows_t, rows_t,
                       pltpu.VMEM((Q * CH, D), jnp.float32),
                       pltpu.SemaphoreType.DMA, pltpu.SemaphoreType.DMA,
                       pltpu.SemaphoreType.DMA, pltpu.SemaphoreType.DMA])
    def k(feat_hbm, idx_hbm, out_hbm, idx_all, r0, r1, r2, r3, out_v,
          s0, s1, s2, s3):
        wid = _worker_id()
        base = wid * PW
        rows = [r0, r1, r2, r3]
        sems = [s0, s1, s2, s3]
        pltpu.sync_copy(idx_hbm.at[pl.ds(base * K, PW * K)], idx_all)

        def iteration(i, carry):
            c0 = i * Q
            copies = []
            for q in range(Q):
                src = feat_hbm.at[idx_all.at[pl.ds((c0 + q) * CH * K, CH * K)]]
                copies.append(pltpu.async_copy(src, rows[q], sems[q]))
            for q in range(Q):
                copies[q].wait()
                rv = rows[q]

                def per_point(p, carry2, _q=q, _rv=rv):
                    for cc in range(D // 16):
                        sl = pl.ds(cc * 16, 16)
                        acc = _rv[p * K, sl]
                        for r in range(1, K):
                            acc = jnp.maximum(acc, _rv[p * K + r, sl])
                        out_v[_q * CH + p, sl] = acc
                    return carry2

                lax.fori_loop(0, CH, per_point, 0)
            pltpu.sync_copy(out_v, out_hbm.at[pl.ds(base + c0 * CH, Q * CH)])
            return carry

        lax.fori_loop(0, NI, iteration, 0)

    return k(feat_flat, idx_flat).reshape(B, N, D)


def _mlp1_body(pts_ref, cov_ref, w1p_ref, w1c_ref, b1_ref, w2_ref, b2_ref,
               w3_ref, b3_ref, out_ref):
    p = pts_ref[0]
    c = cov_ref[0]
    h = (lax.dot_general(p, w1p_ref[...], (((1,), (1,)), ((), ())),
                         preferred_element_type=jnp.float32)
         + lax.dot_general(c, w1c_ref[...], (((1,), (1,)), ((), ())),
                           preferred_element_type=jnp.float32))
    h = jax.nn.relu(h + b1_ref[...])
    h = jax.nn.relu(lax.dot_general(h, w2_ref[...], (((1,), (1,)), ((), ())),
                                    preferred_element_type=jnp.float32) + b2_ref[...])
    h = jax.nn.relu(lax.dot_general(h, w3_ref[...], (((1,), (1,)), ((), ())),
                                    preferred_element_type=jnp.float32) + b3_ref[...])
    out_ref[0] = h


def _mlp1(pts, cov9, w1, b1, w2, b2, w3, b3):
    w1p, w1c = w1[:, :3], w1[:, 3:]
    full = lambda shape: pl.BlockSpec(shape, lambda b, i: tuple(0 for _ in shape))
    return pl.pallas_call(
        _mlp1_body,
        grid=(B, N // NB),
        in_specs=[pl.BlockSpec((1, NB, 3), lambda b, i: (b, i, 0)),
                  pl.BlockSpec((1, NB, 9), lambda b, i: (b, i, 0)),
                  full((64, 3)), full((64, 9)), full((64,)),
                  full((64, 64)), full((64,)), full((64, 64)), full((64,))],
        out_specs=pl.BlockSpec((1, NB, 64), lambda b, i: (b, i, 0)),
        out_shape=jax.ShapeDtypeStruct((B, N, 64), jnp.float32),
    )(pts, cov9, w1p, w1c, b1, w2, b2, w3, b3)


def _lin_conv_body(x_ref, lw_ref, lb_ref, cw_ref, cb_ref, out_ref):
    x = x_ref[0]
    t = lax.dot_general(x, lw_ref[...], (((1,), (1,)), ((), ())),
                        preferred_element_type=jnp.float32) + lb_ref[...]
    h = jax.nn.relu(lax.dot_general(t, cw_ref[...], (((1,), (1,)), ((), ())),
                                    preferred_element_type=jnp.float32) + cb_ref[...])
    out_ref[0] = h


def _lin_conv(x, lw, lb, cw, cb):
    Din, Dout = lw.shape[1], cw.shape[0]
    full = lambda shape: pl.BlockSpec(shape, lambda b, i: tuple(0 for _ in shape))
    return pl.pallas_call(
        _lin_conv_body,
        grid=(B, N // NB),
        in_specs=[pl.BlockSpec((1, NB, Din), lambda b, i: (b, i, 0)),
                  full(lw.shape), full(lb.shape), full(cw.shape), full(cb.shape)],
        out_specs=pl.BlockSpec((1, NB, Dout), lambda b, i: (b, i, 0)),
        out_shape=jax.ShapeDtypeStruct((B, N, Dout), jnp.float32),
    )(x, lw, lb, cw, cb)


def _tail_body(x_ref, lw_ref, lb_ref, cw_ref, cb_ref, m1w_ref, m1b_ref,
               m2w_ref, m2b_ref, emb_ref, acc_ref):
    j = pl.program_id(1)
    x = x_ref[0]
    t = lax.dot_general(x, lw_ref[...], (((1,), (1,)), ((), ())),
                        preferred_element_type=jnp.float32) + lb_ref[...]
    y = lax.dot_general(t, cw_ref[...], (((1,), (1,)), ((), ())),
                        preferred_element_type=jnp.float32) + cb_ref[...]
    bm = jnp.max(y, axis=0, keepdims=True)

    @pl.when(j == 0)
    def _():
        acc_ref[...] = bm

    @pl.when(j > 0)
    def _():
        acc_ref[...] = jnp.maximum(acc_ref[...], bm)

    @pl.when(j == pl.num_programs(1) - 1)
    def _():
        g = acc_ref[...]
        g1 = jax.nn.relu(
            lax.dot_general(g, m1w_ref[...], (((1,), (1,)), ((), ())),
                            preferred_element_type=jnp.float32) + m1b_ref[...])
        g2 = lax.dot_general(g1, m2w_ref[...], (((1,), (1,)), ((), ())),
                             preferred_element_type=jnp.float32) + m2b_ref[...]
        emb_ref[...] = g2[None]


def _tail(x, lw, lb, cw, cb, m1w, m1b, m2w, m2b):
    full = lambda shape: pl.BlockSpec(shape, lambda b, i: tuple(0 for _ in shape))
    return pl.pallas_call(
        _tail_body,
        grid=(B, N // NB),
        in_specs=[pl.BlockSpec((1, NB, 128), lambda b, i: (b, i, 0)),
                  full(lw.shape), full(lb.shape), full(cw.shape), full(cb.shape),
                  full(m1w.shape), full(m1b.shape), full(m2w.shape), full(m2b.shape)],
        out_specs=pl.BlockSpec((1, 1, 512), lambda b, i: (b, 0, 0)),
        out_shape=jax.ShapeDtypeStruct((B, 1, 512), jnp.float32),
        scratch_shapes=[pltpu.VMEM((1, 1024), jnp.float32)],
    )(x, lw, lb, cw, cb, m1w, m1b, m2w, m2b)


def _dec_body(emb_ref, grid_ref, w1g_ref, e1w_ref, b1_ref, w2_ref, b2_ref,
              w3_ref, b3_ref, v1f_ref, e2w_ref, c1_ref, v2_ref, c2_ref,
              v3_ref, c3_ref, out_ref, fold1_ref):
    emb = emb_ref[0]
    e1 = lax.dot_general(emb, e1w_ref[...], (((1,), (1,)), ((), ())),
                         preferred_element_type=jnp.float32)
    e2 = lax.dot_general(emb, e2w_ref[...], (((1,), (1,)), ((), ())),
                         preferred_element_type=jnp.float32)
    g = grid_ref[...]
    t = jax.nn.relu(
        lax.dot_general(g, w1g_ref[...], (((1,), (1,)), ((), ())),
                        preferred_element_type=jnp.float32) + e1 + b1_ref[...])
    t = jax.nn.relu(lax.dot_general(t, w2_ref[...], (((1,), (1,)), ((), ())),
                                    preferred_element_type=jnp.float32) + b2_ref[...])
    f1 = lax.dot_general(t, w3_ref[...], (((1,), (1,)), ((), ())),
                         preferred_element_type=jnp.float32) + b3_ref[...]
    u = jax.nn.relu(
        lax.dot_general(f1, v1f_ref[...], (((1,), (1,)), ((), ())),
                        preferred_element_type=jnp.float32) + e2 + c1_ref[...])
    u = jax.nn.relu(lax.dot_general(u, v2_ref[...], (((1,), (1,)), ((), ())),
                                    preferred_element_type=jnp.float32) + c2_ref[...])
    f2 = lax.dot_general(u, v3_ref[...], (((1,), (1,)), ((), ())),
                         preferred_element_type=jnp.float32) + c3_ref[...]
    fold1_ref[0] = f1
    out_ref[0] = f2


@functools.lru_cache(maxsize=1)
def _grid_const():
    xs = np.linspace(-3, 3, 45)
    pts = np.array(list(itertools.product(xs, xs)), dtype=np.float32)
    return pts


def _decoder(emb, f1_w1, f1_b1, f1_w2, f1_b2, f1_w3, f1_b3,
             f2_w1, f2_b1, f2_w2, f2_b2, f2_w3, f2_b3):
    gridc = jnp.asarray(_grid_const())
    w1g = f1_w1[:, 512:514]
    e1w = f1_w1[:, :512]
    v1f = f2_w1[:, 512:515]
    e2w = f2_w1[:, :512]
    full = lambda shape: pl.BlockSpec(shape, lambda b, i: tuple(0 for _ in shape))
    return pl.pallas_call(
        _dec_body,
        grid=(B, (M + MB - 1) // MB),
        in_specs=[pl.BlockSpec((1, 1, 512), lambda b, i: (b, 0, 0)),
                  pl.BlockSpec((MB, 2), lambda b, i: (i, 0)),
                  full(w1g.shape), full(e1w.shape), full(f1_b1.shape),
                  full(f1_w2.shape), full(f1_b2.shape),
                  full(f1_w3.shape), full(f1_b3.shape),
                  full(v1f.shape), full(e2w.shape), full(f2_b1.shape),
                  full(f2_w2.shape), full(f2_b2.shape),
                  full(f2_w3.shape), full(f2_b3.shape)],
        out_specs=[pl.BlockSpec((1, MB, 3), lambda b, i: (b, i, 0)),
                   pl.BlockSpec((1, MB, 3), lambda b, i: (b, i, 0))],
        out_shape=[jax.ShapeDtypeStruct((B, M, 3), jnp.float32),
                   jax.ShapeDtypeStruct((B, M, 3), jnp.float32)],
    )(emb, gridc, w1g, e1w, f1_b1, f1_w2, f1_b2, f1_w3, f1_b3,
      v1f, e2w, f2_b1, f2_w2, f2_b2, f2_w3, f2_b3)


def kernel(pts, mlp1_w1, mlp1_b1, mlp1_w2, mlp1_b2, mlp1_w3, mlp1_b3,
           lin1_w, lin1_b, conv1_w, conv1_b, lin2_w, lin2_b,
           conv2_w, conv2_b, mlp2_w1, mlp2_b1, mlp2_w2, mlp2_b2,
           f1_w1, f1_b1, f1_w2, f1_b2, f1_w3, f1_b3,
           f2_w1, f2_b1, f2_w2, f2_b2, f2_w3, f2_b3):
    idxf, cov9 = _knn(pts)
    h1 = _mlp1(pts, cov9, mlp1_w1, mlp1_b1, mlp1_w2, mlp1_b2, mlp1_w3, mlp1_b3)
    m1 = _maxpool(h1, idxf)
    h2 = _lin_conv(m1, lin1_w, lin1_b, conv1_w, conv1_b)
    m2 = _maxpool(h2, idxf)
    feat = _tail(m2, lin2_w, lin2_b, conv2_w, conv2_b,
                 mlp2_w1, mlp2_b1, mlp2_w2, mlp2_b2)
    output, fold1 = _decoder(feat, f1_w1, f1_b1, f1_w2, f1_b2, f1_w3, f1_b3,
                             f2_w1, f2_b1, f2_w2, f2_b2, f2_w3, f2_b3)
    return output, feat, feat.reshape(B, 512), fold1
